# baseline (speedup 1.0000x reference)
import functools

import jax
import jax.numpy as jnp
from jax import lax
from jax.experimental import pallas as pl
from jax.experimental.pallas import tpu as pltpu
from jax.experimental.pallas import tpu_sc as plsc

N = 10000
E = 160000
D = 256
DE = 16
H = 256
HALF = 128
NTILES = 16
EPT = E // (2 * NTILES) * 2
CHUNK = 80
NCHUNK = 10000 // CHUNK
GROUPS = HALF // 16


def _pre_body(x_ref, wxs_ref, wxd_ref, xs0_ref, xs1_ref, xd0_ref, xd1_ref):
    xb = x_ref[:]
    ps = jnp.dot(xb, wxs_ref[:], preferred_element_type=jnp.float32)
    pd = jnp.dot(xb, wxd_ref[:], preferred_element_type=jnp.float32)
    xs0_ref[:] = ps[:, :HALF]
    xs1_ref[:] = ps[:, HALF:]
    xd0_ref[:] = pd[:, :HALF]
    xd1_ref[:] = pd[:, HALF:]


def _pre(x, wxs, wxd):
    mb = 1000
    grid = (N // mb,)
    out = jax.ShapeDtypeStruct((N, HALF), jnp.float32)
    return pl.pallas_call(
        _pre_body,
        grid=grid,
        in_specs=[
            pl.BlockSpec((mb, D), lambda i: (i, 0)),
            pl.BlockSpec((D, H), lambda i: (0, 0)),
            pl.BlockSpec((D, H), lambda i: (0, 0)),
        ],
        out_specs=[pl.BlockSpec((mb, HALF), lambda i: (i, 0))] * 4,
        out_shape=[out, out, out, out],
    )(x, wxs, wxd)


def _ea_body(ein_ref, wea_ref, be1_ref, ea0_ref, ea1_ref):
    r = jnp.dot(ein_ref[:], wea_ref[:], preferred_element_type=jnp.float32)
    r = r + be1_ref[:]
    ea0_ref[:] = r[:, :HALF]
    ea1_ref[:] = r[:, HALF:]


def _ea(edge_attr, wea, be1):
    eb = 2000
    grid = (E // eb,)
    out = jax.ShapeDtypeStruct((E, HALF), jnp.float32)
    return pl.pallas_call(
        _ea_body,
        grid=grid,
        in_specs=[
            pl.BlockSpec((eb, DE), lambda i: (i, 0)),
            pl.BlockSpec((DE, H), lambda i: (0, 0)),
            pl.BlockSpec((1, H), lambda i: (0, 0)),
        ],
        out_specs=[pl.BlockSpec((eb, HALF), lambda i: (i, 0))] * 2,
        out_shape=[out, out],
    )(edge_attr, wea, be1)


def _post_body(x_ref, agga_ref, aggb_ref, w1x_ref, w1a_ref, w1b_ref, bn1_ref,
               w2_ref, bn2_ref, out_ref):
    acc = jnp.dot(x_ref[:], w1x_ref[:], preferred_element_type=jnp.float32)
    acc += jnp.dot(agga_ref[:], w1a_ref[:], preferred_element_type=jnp.float32)
    acc += jnp.dot(aggb_ref[:], w1b_ref[:], preferred_element_type=jnp.float32)
    hn = jnp.maximum(acc + bn1_ref[:], 0.0)
    out_ref[:] = jnp.dot(hn, w2_ref[:], preferred_element_type=jnp.float32) + bn2_ref[:]


def _post(x, agga, aggb, w1x, w1a, w1b, bn1, w2p, bn2p):
    mb = 1000
    grid = (N // mb,)
    return pl.pallas_call(
        _post_body,
        grid=grid,
        in_specs=[
            pl.BlockSpec((mb, D), lambda i: (i, 0)),
            pl.BlockSpec((mb, HALF), lambda i: (i, 0)),
            pl.BlockSpec((mb, HALF), lambda i: (i, 0)),
            pl.BlockSpec((D, H), lambda i: (0, 0)),
            pl.BlockSpec((HALF, H), lambda i: (0, 0)),
            pl.BlockSpec((HALF, H), lambda i: (0, 0)),
            pl.BlockSpec((1, H), lambda i: (0, 0)),
            pl.BlockSpec((H, HALF), lambda i: (0, 0)),
            pl.BlockSpec((1, HALF), lambda i: (0, 0)),
        ],
        out_specs=pl.BlockSpec((mb, HALF), lambda i: (i, 0)),
        out_shape=jax.ShapeDtypeStruct((N, HALF), jnp.float32),
    )(x, agga, aggb, w1x, w1a, w1b, bn1, w2p, bn2p)


def _hsum(a):
    lanes = lax.iota(jnp.int32, 16)
    for sh in (8, 4, 2, 1):
        a = a + jnp.take(a, lanes ^ sh)
    return a

def _sc_core_loop(core, s, xs_hbm, xd_hbm, ea_hbm, src_r, dst_r, w2t_hbm,
                  out_agg, out_ep,
                  idx_src, idx_dst, rows_a, rows_b, h_v, ep0_v, ep1_v,
                  w0_v, w1_v, agg_sh, sem0, sem1):
    pltpu.sync_copy(w2t_hbm.at[0, pl.ds(core * HALF, HALF)], w0_v)
    pltpu.sync_copy(w2t_hbm.at[1, pl.ds(core * HALF, HALF)], w1_v)
    w0s = [w0_v[pl.ds(g * 16, 16)] for g in range(GROUPS)]
    w1s = [w1_v[pl.ds(g * 16, 16)] for g in range(GROUPS)]
    lanes = lax.iota(jnp.int32, 16)
    zero16 = jnp.zeros((16,), jnp.float32)

    def chunk_body(k, _):
        pltpu.sync_copy(src_r.at[s, k], idx_src)
        pltpu.sync_copy(dst_r.at[s, k], idx_dst)
        cp_a = pltpu.async_copy(xs_hbm.at[idx_src.at[0]], rows_a, sem0)
        cp_b = pltpu.async_copy(xd_hbm.at[idx_dst.at[0]], rows_b, sem1)
        pltpu.sync_copy(ea_hbm.at[pl.ds(s * EPT + k * CHUNK, CHUNK)], h_v)
        cp_a.wait()
        cp_b.wait()

        def block_body(j, _):
            def edge_body(t, carry):
                r0, r1 = carry
                e = j * 16 + t
                acc0 = zero16
                acc1 = zero16
                for g in range(GROUPS):
                    sl = pl.ds(g * 16, 16)
                    h = jnp.maximum(
                        rows_a[e, sl] + rows_b[e, sl] + h_v[e, sl], 0.0)
                    h_v[e, sl] = h
                    acc0 = acc0 + h * w0s[g]
                    acc1 = acc1 + h * w1s[g]
                sel = lanes == t
                return (jnp.where(sel, _hsum(acc0), r0),
                        jnp.where(sel, _hsum(acc1), r1))

            r0, r1 = lax.fori_loop(0, 16, edge_body, (zero16, zero16))
            ep0_v[0, pl.ds(j * 16, 16)] = r0
            ep1_v[0, pl.ds(j * 16, 16)] = r1
            return 0

        lax.fori_loop(0, CHUNK // 16, block_body, 0)
        pltpu.sync_copy(ep0_v, out_ep.at[core, 0, s, k])
        pltpu.sync_copy(ep1_v, out_ep.at[core, 1, s, k])
        pltpu.sync_copy(h_v, agg_sh.at[idx_dst.at[0]], add=True)
        return 0

    lax.fori_loop(0, NCHUNK, chunk_body, 0)


def _sc_body(xs0, xs1, xd0, xd1, ea0, ea1, src_r, dst_r, w2t, zeros,
             out_agg, out_ep,
             idx_src, idx_dst, rows_a, rows_b, h_v, ep0_v, ep1_v,
             w0_v, w1_v, agg_sh, sem0, sem1):
    c = lax.axis_index("c")
    s = lax.axis_index("s")

    @pl.when(s < 10)
    def _():
        pltpu.sync_copy(zeros, agg_sh.at[pl.ds(s * 1000, 1000)])

    plsc.subcore_barrier()

    @pl.when(c == 0)
    def _():
        _sc_core_loop(0, s, xs0, xd0, ea0, src_r, dst_r, w2t,
                      out_agg, out_ep, idx_src, idx_dst, rows_a, rows_b,
                      h_v, ep0_v, ep1_v, w0_v, w1_v, agg_sh, sem0, sem1)

    @pl.when(c == 1)
    def _():
        _sc_core_loop(1, s, xs1, xd1, ea1, src_r, dst_r, w2t,
                      out_agg, out_ep, idx_src, idx_dst, rows_a, rows_b,
                      h_v, ep0_v, ep1_v, w0_v, w1_v, agg_sh, sem0, sem1)

    plsc.subcore_barrier()

    @pl.when(jnp.logical_and(c == 0, s < 10))
    def _():
        pltpu.sync_copy(agg_sh.at[pl.ds(s * 1000, 1000)],
                        out_agg.at[0, pl.ds(s * 1000, 1000)])

    @pl.when(jnp.logical_and(c == 1, s < 10))
    def _():
        pltpu.sync_copy(agg_sh.at[pl.ds(s * 1000, 1000)],
                        out_agg.at[1, pl.ds(s * 1000, 1000)])


def _sc_call(xs0, xs1, xd0, xd1, ea0, ea1, src_r, dst_r, w2t, zeros):
    mesh = plsc.VectorSubcoreMesh(core_axis_name="c", subcore_axis_name="s")
    fn = pl.kernel(
        _sc_body,
        out_type=[
            jax.ShapeDtypeStruct((2, N, HALF), jnp.float32),
            jax.ShapeDtypeStruct((2, 2, NTILES, NCHUNK, 1, CHUNK),
                                 jnp.float32),
        ],
        mesh=mesh,
        scratch_types=[
            pltpu.VMEM((1, CHUNK), jnp.int32),
            pltpu.VMEM((1, CHUNK), jnp.int32),
            pltpu.VMEM((CHUNK, HALF), jnp.float32),
            pltpu.VMEM((CHUNK, HALF), jnp.float32),
            pltpu.VMEM((CHUNK, HALF), jnp.float32),
            pltpu.VMEM((1, CHUNK), jnp.float32),
            pltpu.VMEM((1, CHUNK), jnp.float32),
            pltpu.VMEM((HALF,), jnp.float32),
            pltpu.VMEM((HALF,), jnp.float32),
            pltpu.VMEM_SHARED((N, HALF), jnp.float32),
            pltpu.SemaphoreType.DMA,
            pltpu.SemaphoreType.DMA,
        ],
    )
    return fn(xs0, xs1, xd0, xd1, ea0, ea1, src_r, dst_r, w2t, zeros)


@jax.jit
def kernel(x, edge_index, edge_attr, We1, be1, We2, be2, Wn1, bn1, Wn2, bn2):
    wxs = We1[:D]
    wxd = We1[D:2 * D]
    wea = We1[2 * D:]

    xs0, xs1, xd0, xd1 = _pre(x, wxs, wxd)
    ea0, ea1 = _ea(edge_attr, wea, be1.reshape(1, H))

    src_r = edge_index[0].reshape(NTILES, NCHUNK, 1, CHUNK)
    dst_r = edge_index[1].reshape(NTILES, NCHUNK, 1, CHUNK)
    w2t = We2.T.reshape(2, H)
    zeros = jnp.zeros((1000, HALF), jnp.float32)

    out_agg, out_ep = _sc_call(xs0, xs1, xd0, xd1, ea0, ea1,
                               src_r, dst_r, w2t, zeros)

    ep = out_ep.reshape(2, 2, E)
    edge_pred = (ep[0] + ep[1]).T + be2

    w1x = Wn1[:D]
    w1a = Wn1[D:D + HALF]
    w1b = Wn1[D + HALF:]
    w2p = jnp.zeros((H, HALF), jnp.float32).at[:, :Wn2.shape[1]].set(Wn2)
    bn2p = jnp.zeros((1, HALF), jnp.float32).at[0, :bn2.shape[0]].set(bn2)
    node_full = _post(x, out_agg[0], out_agg[1], w1x, w1a, w1b,
                      bn1.reshape(1, H), w2p, bn2p)
    node_pred = node_full[:, :Wn2.shape[1]]
    return node_pred, edge_pred


# speedup vs baseline: 1.0846x; 1.0846x over previous
import functools

import jax
import jax.numpy as jnp
from jax import lax
from jax.experimental import pallas as pl
from jax.experimental.pallas import tpu as pltpu
from jax.experimental.pallas import tpu_sc as plsc

N = 10000
E = 160000
D = 256
DE = 16
H = 256
HALF = 128
NTILES = 16
EPT = E // (2 * NTILES) * 2
CHUNK = 80
NCHUNK = 10000 // CHUNK
GROUPS = HALF // 16


def _pre_body(x_ref, wxs_ref, wxd_ref, xs0_ref, xs1_ref, xd0_ref, xd1_ref):
    xb = x_ref[:]
    ps = jnp.dot(xb, wxs_ref[:], preferred_element_type=jnp.float32)
    pd = jnp.dot(xb, wxd_ref[:], preferred_element_type=jnp.float32)
    xs0_ref[:] = ps[:, :HALF]
    xs1_ref[:] = ps[:, HALF:]
    xd0_ref[:] = pd[:, :HALF]
    xd1_ref[:] = pd[:, HALF:]


def _pre(x, wxs, wxd):
    mb = 1000
    grid = (N // mb,)
    out = jax.ShapeDtypeStruct((N, HALF), jnp.float32)
    return pl.pallas_call(
        _pre_body,
        grid=grid,
        in_specs=[
            pl.BlockSpec((mb, D), lambda i: (i, 0)),
            pl.BlockSpec((D, H), lambda i: (0, 0)),
            pl.BlockSpec((D, H), lambda i: (0, 0)),
        ],
        out_specs=[pl.BlockSpec((mb, HALF), lambda i: (i, 0))] * 4,
        out_shape=[out, out, out, out],
    )(x, wxs, wxd)


def _ea_body(ein_ref, wea_ref, be1_ref, ea0_ref, ea1_ref):
    r = jnp.dot(ein_ref[:], wea_ref[:], preferred_element_type=jnp.float32)
    r = r + be1_ref[:]
    ea0_ref[:] = r[:, :HALF]
    ea1_ref[:] = r[:, HALF:]


def _ea(edge_attr, wea, be1):
    eb = 2000
    grid = (E // eb,)
    out = jax.ShapeDtypeStruct((E, HALF), jnp.float32)
    return pl.pallas_call(
        _ea_body,
        grid=grid,
        in_specs=[
            pl.BlockSpec((eb, DE), lambda i: (i, 0)),
            pl.BlockSpec((DE, H), lambda i: (0, 0)),
            pl.BlockSpec((1, H), lambda i: (0, 0)),
        ],
        out_specs=[pl.BlockSpec((eb, HALF), lambda i: (i, 0))] * 2,
        out_shape=[out, out],
    )(edge_attr, wea, be1)


def _post_body(x_ref, agga_ref, aggb_ref, w1x_ref, w1a_ref, w1b_ref, bn1_ref,
               w2_ref, bn2_ref, out_ref):
    acc = jnp.dot(x_ref[:], w1x_ref[:], preferred_element_type=jnp.float32)
    acc += jnp.dot(agga_ref[:], w1a_ref[:], preferred_element_type=jnp.float32)
    acc += jnp.dot(aggb_ref[:], w1b_ref[:], preferred_element_type=jnp.float32)
    hn = jnp.maximum(acc + bn1_ref[:], 0.0)
    out_ref[:] = jnp.dot(hn, w2_ref[:], preferred_element_type=jnp.float32) + bn2_ref[:]


def _post(x, agga, aggb, w1x, w1a, w1b, bn1, w2p, bn2p):
    mb = 1000
    grid = (N // mb,)
    return pl.pallas_call(
        _post_body,
        grid=grid,
        in_specs=[
            pl.BlockSpec((mb, D), lambda i: (i, 0)),
            pl.BlockSpec((mb, HALF), lambda i: (i, 0)),
            pl.BlockSpec((mb, HALF), lambda i: (i, 0)),
            pl.BlockSpec((D, H), lambda i: (0, 0)),
            pl.BlockSpec((HALF, H), lambda i: (0, 0)),
            pl.BlockSpec((HALF, H), lambda i: (0, 0)),
            pl.BlockSpec((1, H), lambda i: (0, 0)),
            pl.BlockSpec((H, HALF), lambda i: (0, 0)),
            pl.BlockSpec((1, HALF), lambda i: (0, 0)),
        ],
        out_specs=pl.BlockSpec((mb, HALF), lambda i: (i, 0)),
        out_shape=jax.ShapeDtypeStruct((N, HALF), jnp.float32),
    )(x, agga, aggb, w1x, w1a, w1b, bn1, w2p, bn2p)


def _hsum(a):
    lanes = lax.iota(jnp.int32, 16)
    for sh in (8, 4, 2, 1):
        a = a + jnp.take(a, lanes ^ sh)
    return a

def _sc_core_loop(core, s, xs_hbm, xd_hbm, ea_hbm, src_r, w2t_hbm,
                  out_agg, out_ep,
                  idx_src, rows_a, rows_b, h_v, ep0_v,
                  w0_v, w1_v, agg_sh, sem0, sem1):
    pltpu.sync_copy(w2t_hbm.at[0, pl.ds(core * HALF, HALF)], w0_v)
    pltpu.sync_copy(w2t_hbm.at[1, pl.ds(core * HALF, HALF)], w1_v)
    w0s = [w0_v[pl.ds(g * 16, 16)] for g in range(GROUPS)]
    w1s = [w1_v[pl.ds(g * 16, 16)] for g in range(GROUPS)]
    lanes = lax.iota(jnp.int32, 16)
    zero16 = jnp.zeros((16,), jnp.float32)

    def chunk_body(k, _):
        pltpu.sync_copy(src_r.at[s, k], idx_src)
        cp_a = pltpu.async_copy(xs_hbm.at[idx_src.at[0]], rows_a, sem0)
        cp_b = pltpu.async_copy(xd_hbm.at[idx_src.at[1]], rows_b, sem1)
        pltpu.sync_copy(ea_hbm.at[pl.ds(s * EPT + k * CHUNK, CHUNK)], h_v)
        cp_a.wait()
        cp_b.wait()

        def block_body(j, _):
            def edge_body(t, carry):
                r0, r1 = carry
                e = j * 16 + t
                acc0 = zero16
                acc1 = zero16
                for g in range(GROUPS):
                    sl = pl.ds(g * 16, 16)
                    h = jnp.maximum(
                        rows_a[e, sl] + rows_b[e, sl] + h_v[e, sl], 0.0)
                    h_v[e, sl] = h
                    acc0 = acc0 + h * w0s[g]
                    acc1 = acc1 + h * w1s[g]
                sel = lanes == t
                return (jnp.where(sel, _hsum(acc0), r0),
                        jnp.where(sel, _hsum(acc1), r1))

            r0, r1 = lax.fori_loop(0, 16, edge_body, (zero16, zero16))
            ep0_v[0, pl.ds(j * 16, 16)] = r0
            ep0_v[1, pl.ds(j * 16, 16)] = r1
            return 0

        lax.fori_loop(0, CHUNK // 16, block_body, 0)
        pltpu.sync_copy(ep0_v, out_ep.at[core, s, k])
        pltpu.sync_copy(h_v, agg_sh.at[idx_src.at[1]], add=True)
        return 0

    lax.fori_loop(0, NCHUNK, chunk_body, 0)


def _sc_body(xs0, xs1, xd0, xd1, ea0, ea1, src_r, w2t, zeros,
             out_agg, out_ep,
             idx_src, rows_a, rows_b, h_v, ep0_v,
             w0_v, w1_v, agg_sh, sem0, sem1):
    c = lax.axis_index("c")
    s = lax.axis_index("s")

    @pl.when(s < 10)
    def _():
        pltpu.sync_copy(zeros, agg_sh.at[pl.ds(s * 1000, 1000)])

    plsc.subcore_barrier()

    @pl.when(c == 0)
    def _():
        _sc_core_loop(0, s, xs0, xd0, ea0, src_r, w2t,
                      out_agg, out_ep, idx_src, rows_a, rows_b,
                      h_v, ep0_v, w0_v, w1_v, agg_sh, sem0, sem1)

    @pl.when(c == 1)
    def _():
        _sc_core_loop(1, s, xs1, xd1, ea1, src_r, w2t,
                      out_agg, out_ep, idx_src, rows_a, rows_b,
                      h_v, ep0_v, w0_v, w1_v, agg_sh, sem0, sem1)

    plsc.subcore_barrier()

    @pl.when(jnp.logical_and(c == 0, s < 10))
    def _():
        pltpu.sync_copy(agg_sh.at[pl.ds(s * 1000, 1000)],
                        out_agg.at[0, pl.ds(s * 1000, 1000)])

    @pl.when(jnp.logical_and(c == 1, s < 10))
    def _():
        pltpu.sync_copy(agg_sh.at[pl.ds(s * 1000, 1000)],
                        out_agg.at[1, pl.ds(s * 1000, 1000)])


def _sc_call(xs0, xs1, xd0, xd1, ea0, ea1, src_r, w2t, zeros):
    mesh = plsc.VectorSubcoreMesh(core_axis_name="c", subcore_axis_name="s")
    fn = pl.kernel(
        _sc_body,
        out_type=[
            jax.ShapeDtypeStruct((2, N, HALF), jnp.float32),
            jax.ShapeDtypeStruct((2, NTILES, NCHUNK, 2, CHUNK),
                                 jnp.float32),
        ],
        mesh=mesh,
        scratch_types=[
            pltpu.VMEM((2, CHUNK), jnp.int32),
            pltpu.VMEM((CHUNK, HALF), jnp.float32),
            pltpu.VMEM((CHUNK, HALF), jnp.float32),
            pltpu.VMEM((CHUNK, HALF), jnp.float32),
            pltpu.VMEM((2, CHUNK), jnp.float32),
            pltpu.VMEM((HALF,), jnp.float32),
            pltpu.VMEM((HALF,), jnp.float32),
            pltpu.VMEM_SHARED((N, HALF), jnp.float32),
            pltpu.SemaphoreType.DMA,
            pltpu.SemaphoreType.DMA,
        ],
    )
    return fn(xs0, xs1, xd0, xd1, ea0, ea1, src_r, w2t, zeros)


@jax.jit
def kernel(x, edge_index, edge_attr, We1, be1, We2, be2, Wn1, bn1, Wn2, bn2):
    wxs = We1[:D]
    wxd = We1[D:2 * D]
    wea = We1[2 * D:]

    xs0, xs1, xd0, xd1 = _pre(x, wxs, wxd)
    ea0, ea1 = _ea(edge_attr, wea, be1.reshape(1, H))

    src_r = edge_index.reshape(2, NTILES, NCHUNK, CHUNK).transpose(1, 2, 0, 3)
    w2t = We2.T.reshape(2, H)
    zeros = jnp.zeros((1000, HALF), jnp.float32)

    out_agg, out_ep = _sc_call(xs0, xs1, xd0, xd1, ea0, ea1,
                               src_r, w2t, zeros)

    ep = jnp.moveaxis(out_ep[0] + out_ep[1], 2, 0).reshape(2, E)
    edge_pred = ep.T + be2

    w1x = Wn1[:D]
    w1a = Wn1[D:D + HALF]
    w1b = Wn1[D + HALF:]
    w2p = jnp.zeros((H, HALF), jnp.float32).at[:, :Wn2.shape[1]].set(Wn2)
    bn2p = jnp.zeros((1, HALF), jnp.float32).at[0, :bn2.shape[0]].set(bn2)
    node_full = _post(x, out_agg[0], out_agg[1], w1x, w1a, w1b,
                      bn1.reshape(1, H), w2p, bn2p)
    node_pred = node_full[:, :Wn2.shape[1]]
    return node_pred, edge_pred


# speedup vs baseline: 1.1419x; 1.0528x over previous
import functools

import jax
import jax.numpy as jnp
from jax import lax
from jax.experimental import pallas as pl
from jax.experimental.pallas import tpu as pltpu
from jax.experimental.pallas import tpu_sc as plsc

N = 10000
E = 160000
D = 256
DE = 16
H = 256
HALF = 128
NTILES = 16
EPT = E // (2 * NTILES) * 2
CHUNK = 80
NCHUNK = 10000 // CHUNK
BLKC = 25
NBLK = NCHUNK // BLKC
GROUPS = HALF // 16


def _pre_body(x_ref, wxs_ref, wxd_ref, xs0_ref, xs1_ref, xd0_ref, xd1_ref):
    xb = x_ref[:]
    ps = jnp.dot(xb, wxs_ref[:], preferred_element_type=jnp.float32)
    pd = jnp.dot(xb, wxd_ref[:], preferred_element_type=jnp.float32)
    xs0_ref[:] = ps[:, :HALF]
    xs1_ref[:] = ps[:, HALF:]
    xd0_ref[:] = pd[:, :HALF]
    xd1_ref[:] = pd[:, HALF:]


def _pre(x, wxs, wxd):
    mb = 1000
    grid = (N // mb,)
    out = jax.ShapeDtypeStruct((N, HALF), jnp.float32)
    return pl.pallas_call(
        _pre_body,
        grid=grid,
        in_specs=[
            pl.BlockSpec((mb, D), lambda i: (i, 0)),
            pl.BlockSpec((D, H), lambda i: (0, 0)),
            pl.BlockSpec((D, H), lambda i: (0, 0)),
        ],
        out_specs=[pl.BlockSpec((mb, HALF), lambda i: (i, 0))] * 4,
        out_shape=[out, out, out, out],
    )(x, wxs, wxd)


def _ea_body(ein_ref, wea_ref, be1_ref, ea0_ref, ea1_ref):
    r = jnp.dot(ein_ref[:], wea_ref[:], preferred_element_type=jnp.float32)
    r = r + be1_ref[:]
    ea0_ref[:] = r[:, :HALF]
    ea1_ref[:] = r[:, HALF:]


def _ea(edge_attr, wea, be1):
    eb = 2000
    grid = (E // eb,)
    out = jax.ShapeDtypeStruct((E, HALF), jnp.float32)
    return pl.pallas_call(
        _ea_body,
        grid=grid,
        in_specs=[
            pl.BlockSpec((eb, DE), lambda i: (i, 0)),
            pl.BlockSpec((DE, H), lambda i: (0, 0)),
            pl.BlockSpec((1, H), lambda i: (0, 0)),
        ],
        out_specs=[pl.BlockSpec((eb, HALF), lambda i: (i, 0))] * 2,
        out_shape=[out, out],
    )(edge_attr, wea, be1)


def _post_body(x_ref, agga_ref, aggb_ref, w1x_ref, w1a_ref, w1b_ref, bn1_ref,
               w2_ref, bn2_ref, out_ref):
    acc = jnp.dot(x_ref[:], w1x_ref[:], preferred_element_type=jnp.float32)
    acc += jnp.dot(agga_ref[:], w1a_ref[:], preferred_element_type=jnp.float32)
    acc += jnp.dot(aggb_ref[:], w1b_ref[:], preferred_element_type=jnp.float32)
    hn = jnp.maximum(acc + bn1_ref[:], 0.0)
    out_ref[:] = jnp.dot(hn, w2_ref[:], preferred_element_type=jnp.float32) + bn2_ref[:]


def _post(x, agga, aggb, w1x, w1a, w1b, bn1, w2p, bn2p):
    mb = 1000
    grid = (N // mb,)
    return pl.pallas_call(
        _post_body,
        grid=grid,
        in_specs=[
            pl.BlockSpec((mb, D), lambda i: (i, 0)),
            pl.BlockSpec((mb, HALF), lambda i: (i, 0)),
            pl.BlockSpec((mb, HALF), lambda i: (i, 0)),
            pl.BlockSpec((D, H), lambda i: (0, 0)),
            pl.BlockSpec((HALF, H), lambda i: (0, 0)),
            pl.BlockSpec((HALF, H), lambda i: (0, 0)),
            pl.BlockSpec((1, H), lambda i: (0, 0)),
            pl.BlockSpec((H, HALF), lambda i: (0, 0)),
            pl.BlockSpec((1, HALF), lambda i: (0, 0)),
        ],
        out_specs=pl.BlockSpec((mb, HALF), lambda i: (i, 0)),
        out_shape=jax.ShapeDtypeStruct((N, HALF), jnp.float32),
    )(x, agga, aggb, w1x, w1a, w1b, bn1, w2p, bn2p)


def _hsum(a):
    lanes = lax.iota(jnp.int32, 16)
    for sh in (8, 4, 2, 1):
        a = a + jnp.take(a, lanes ^ sh)
    return a

def _sc_core_loop(core, s, xs_hbm, xd_hbm, ea_hbm, src_r, w2t_hbm,
                  out_agg, out_ep,
                  idx_src, rows_a, rows_b, h_v, ep0_v,
                  w0_v, w1_v, agg_sh, sem0, sem1):
    pltpu.sync_copy(w2t_hbm.at[0, pl.ds(core * HALF, HALF)], w0_v)
    pltpu.sync_copy(w2t_hbm.at[1, pl.ds(core * HALF, HALF)], w1_v)
    w0s = [w0_v[pl.ds(g * 16, 16)] for g in range(GROUPS)]
    w1s = [w1_v[pl.ds(g * 16, 16)] for g in range(GROUPS)]
    lanes = lax.iota(jnp.int32, 16)
    zero16 = jnp.zeros((16,), jnp.float32)

    def chunk_body(k, _):
        blk = k // BLKC
        lc = k - blk * BLKC

        @pl.when(lc == 0)
        def _():
            pltpu.sync_copy(src_r.at[s, blk], idx_src)

        cp_a = pltpu.async_copy(xs_hbm.at[idx_src.at[2 * lc]], rows_a, sem0)
        cp_b = pltpu.async_copy(xd_hbm.at[idx_src.at[2 * lc + 1]], rows_b,
                                sem1)
        pltpu.sync_copy(ea_hbm.at[pl.ds(s * EPT + k * CHUNK, CHUNK)], h_v)
        cp_a.wait()
        cp_b.wait()

        def block_body(j, _):
            def edge_body(t, carry):
                r0, r1 = carry
                e = j * 16 + t
                acc0 = zero16
                acc1 = zero16
                for g in range(GROUPS):
                    sl = pl.ds(g * 16, 16)
                    h = jnp.maximum(
                        rows_a[e, sl] + rows_b[e, sl] + h_v[e, sl], 0.0)
                    h_v[e, sl] = h
                    acc0 = acc0 + h * w0s[g]
                    acc1 = acc1 + h * w1s[g]
                sel = lanes == t
                return (jnp.where(sel, _hsum(acc0), r0),
                        jnp.where(sel, _hsum(acc1), r1))

            r0, r1 = lax.fori_loop(0, 16, edge_body, (zero16, zero16))
            ep0_v[2 * lc, pl.ds(j * 16, 16)] = r0
            ep0_v[2 * lc + 1, pl.ds(j * 16, 16)] = r1
            return 0

        lax.fori_loop(0, CHUNK // 16, block_body, 0)
        pltpu.sync_copy(h_v, agg_sh.at[idx_src.at[2 * lc + 1]], add=True)

        @pl.when(lc == BLKC - 1)
        def _():
            pltpu.sync_copy(ep0_v, out_ep.at[core, s, blk])

        return 0

    lax.fori_loop(0, NCHUNK, chunk_body, 0)


def _sc_body(xs0, xs1, xd0, xd1, ea0, ea1, src_r, w2t, zeros,
             out_agg, out_ep,
             idx_src, rows_a, rows_b, h_v, ep0_v,
             w0_v, w1_v, agg_sh, sem0, sem1):
    c = lax.axis_index("c")
    s = lax.axis_index("s")

    @pl.when(s < 10)
    def _():
        pltpu.sync_copy(zeros, agg_sh.at[pl.ds(s * 1000, 1000)])

    plsc.subcore_barrier()

    @pl.when(c == 0)
    def _():
        _sc_core_loop(0, s, xs0, xd0, ea0, src_r, w2t,
                      out_agg, out_ep, idx_src, rows_a, rows_b,
                      h_v, ep0_v, w0_v, w1_v, agg_sh, sem0, sem1)

    @pl.when(c == 1)
    def _():
        _sc_core_loop(1, s, xs1, xd1, ea1, src_r, w2t,
                      out_agg, out_ep, idx_src, rows_a, rows_b,
                      h_v, ep0_v, w0_v, w1_v, agg_sh, sem0, sem1)

    plsc.subcore_barrier()

    @pl.when(jnp.logical_and(c == 0, s < 10))
    def _():
        pltpu.sync_copy(agg_sh.at[pl.ds(s * 1000, 1000)],
                        out_agg.at[0, pl.ds(s * 1000, 1000)])

    @pl.when(jnp.logical_and(c == 1, s < 10))
    def _():
        pltpu.sync_copy(agg_sh.at[pl.ds(s * 1000, 1000)],
                        out_agg.at[1, pl.ds(s * 1000, 1000)])


def _sc_call(xs0, xs1, xd0, xd1, ea0, ea1, src_r, w2t, zeros):
    mesh = plsc.VectorSubcoreMesh(core_axis_name="c", subcore_axis_name="s")
    fn = pl.kernel(
        _sc_body,
        out_type=[
            jax.ShapeDtypeStruct((2, N, HALF), jnp.float32),
            jax.ShapeDtypeStruct((2, NTILES, NBLK, 2 * BLKC, CHUNK),
                                 jnp.float32),
        ],
        mesh=mesh,
        scratch_types=[
            pltpu.VMEM((2 * BLKC, CHUNK), jnp.int32),
            pltpu.VMEM((CHUNK, HALF), jnp.float32),
            pltpu.VMEM((CHUNK, HALF), jnp.float32),
            pltpu.VMEM((CHUNK, HALF), jnp.float32),
            pltpu.VMEM((2 * BLKC, CHUNK), jnp.float32),
            pltpu.VMEM((HALF,), jnp.float32),
            pltpu.VMEM((HALF,), jnp.float32),
            pltpu.VMEM_SHARED((N, HALF), jnp.float32),
            pltpu.SemaphoreType.DMA,
            pltpu.SemaphoreType.DMA,
        ],
    )
    return fn(xs0, xs1, xd0, xd1, ea0, ea1, src_r, w2t, zeros)


@jax.jit
def kernel(x, edge_index, edge_attr, We1, be1, We2, be2, Wn1, bn1, Wn2, bn2):
    wxs = We1[:D]
    wxd = We1[D:2 * D]
    wea = We1[2 * D:]

    xs0, xs1, xd0, xd1 = _pre(x, wxs, wxd)
    ea0, ea1 = _ea(edge_attr, wea, be1.reshape(1, H))

    src_r = (edge_index.reshape(2, NTILES, NBLK, BLKC, CHUNK)
             .transpose(1, 2, 3, 0, 4).reshape(NTILES, NBLK, 2 * BLKC, CHUNK))
    w2t = We2.T.reshape(2, H)
    zeros = jnp.zeros((1000, HALF), jnp.float32)

    out_agg, out_ep = _sc_call(xs0, xs1, xd0, xd1, ea0, ea1,
                               src_r, w2t, zeros)

    epsum = (out_ep[0] + out_ep[1]).reshape(NTILES, NBLK, BLKC, 2, CHUNK)
    ep = jnp.moveaxis(epsum, 3, 0).reshape(2, E)
    edge_pred = ep.T + be2

    w1x = Wn1[:D]
    w1a = Wn1[D:D + HALF]
    w1b = Wn1[D + HALF:]
    w2p = jnp.zeros((H, HALF), jnp.float32).at[:, :Wn2.shape[1]].set(Wn2)
    bn2p = jnp.zeros((1, HALF), jnp.float32).at[0, :bn2.shape[0]].set(bn2)
    node_full = _post(x, out_agg[0], out_agg[1], w1x, w1a, w1b,
                      bn1.reshape(1, H), w2p, bn2p)
    node_pred = node_full[:, :Wn2.shape[1]]
    return node_pred, edge_pred
